# baseline (speedup 1.0000x reference)
import functools

import jax
import jax.numpy as jnp
from jax import lax
from jax.experimental import pallas as pl
from jax.experimental.pallas import tpu as pltpu

B = 4096
P = 2
L_SENT = 200
D = 64

B_BLK = 2048
NB = B // B_BLK


def _tc_body(idx_ref, tab_ref, out_ref):
    i = pl.program_id(1)

    @pl.when(i == 0)
    def _init():
        out_ref[...] = jnp.zeros_like(out_ref)

    tab = tab_ref[0]
    lvals = idx_ref[...]
    for p in range(P):
        sel = lvals[p, :][None, :] == i
        out_ref[p] = jnp.where(sel, tab, out_ref[p])


def _tc_gather(table_t, idx_t):
    return pl.pallas_call(
        _tc_body,
        grid=(NB, L_SENT),
        in_specs=[
            pl.BlockSpec((P, B_BLK), lambda j, i: (0, j)),
            pl.BlockSpec((1, D, B_BLK), lambda j, i: (i, 0, j)),
        ],
        out_specs=pl.BlockSpec((P, D, B_BLK), lambda j, i: (0, 0, j)),
        out_shape=jax.ShapeDtypeStruct((P, D, B), jnp.float32),
    )(idx_t, table_t)


def kernel(sentence_matrix, entity_pos_index):
    table_t = jnp.transpose(sentence_matrix, (1, 2, 0))
    idx_t = entity_pos_index.astype(jnp.int32).T
    out_t = _tc_gather(table_t, idx_t)
    return jnp.transpose(out_t, (2, 0, 1))


# speedup vs baseline: 1.5862x; 1.5862x over previous
import functools

import jax
import jax.numpy as jnp
from jax import lax
from jax.experimental import pallas as pl
from jax.experimental.pallas import tpu as pltpu

B = 4096
P = 2
L_SENT = 200
D = 64

B_BLK = 4096
NB = B // B_BLK


def _tc_body(idx_ref, tab_ref, out_ref):
    i = pl.program_id(1)

    @pl.when(i == 0)
    def _init():
        out_ref[...] = jnp.zeros_like(out_ref)

    tab = tab_ref[0]
    lvals = idx_ref[...]
    for p in range(P):
        sel = lvals[p, :][None, :] == i
        out_ref[p] = jnp.where(sel, tab, out_ref[p])


def _tc_gather(table_t, idx_t):
    return pl.pallas_call(
        _tc_body,
        grid=(NB, L_SENT),
        in_specs=[
            pl.BlockSpec((P, B_BLK), lambda j, i: (0, j)),
            pl.BlockSpec((1, D, B_BLK), lambda j, i: (i, 0, j)),
        ],
        out_specs=pl.BlockSpec((P, D, B_BLK), lambda j, i: (0, 0, j)),
        out_shape=jax.ShapeDtypeStruct((P, D, B), jnp.float32),
    )(idx_t, table_t)


def kernel(sentence_matrix, entity_pos_index):
    table_t = jnp.transpose(sentence_matrix, (1, 2, 0))
    idx_t = entity_pos_index.astype(jnp.int32).T
    out_t = _tc_gather(table_t, idx_t)
    return jnp.transpose(out_t, (2, 0, 1))
